# baseline (speedup 1.0000x reference)
import functools

import jax
import jax.numpy as jnp
from jax import lax
from jax.experimental import pallas as pl
from jax.experimental.pallas import tpu as pltpu
from jax.experimental.pallas import tpu_sc as plsc

HIDDEN = 1024
EPS = 1e-12

_NUM_CORES = 2
_NUM_SUBCORES = 16
_NW = _NUM_CORES * _NUM_SUBCORES
_LANES = 16


def _rsqrt16(s):
    t = jnp.full((_LANES,), 1.0, jnp.float32)
    for _ in range(24):
        t = 0.5 * (t + s / t)
    return 1.0 / t


@functools.lru_cache(maxsize=None)
def _make_fused(n_tokens, v, h):
    bpw = n_tokens // _NW
    ngrp = bpw // _LANES
    nsl = h // _LANES
    mesh = plsc.VectorSubcoreMesh(core_axis_name="c", subcore_axis_name="s")

    @functools.partial(
        pl.kernel,
        mesh=mesh,
        out_type=jax.ShapeDtypeStruct((n_tokens * h,), jnp.float32),
        scratch_types=[
            pltpu.VMEM((bpw,), jnp.int32),
            pltpu.VMEM((v * h,), jnp.float32),
            pltpu.VMEM((v * h,), jnp.float32),
            pltpu.VMEM((h,), jnp.float32),
            pltpu.VMEM((h,), jnp.float32),
            pltpu.SemaphoreType.DMA,
        ],
    )
    def k(tab_hbm, g_hbm, b_hbm, idx_hbm, out_hbm, idx_v, tab_v, nrm_v, g_v, b_v, sem):
        wid = lax.axis_index("s") * _NUM_CORES + lax.axis_index("c")
        base = wid * bpw
        pltpu.sync_copy(tab_hbm, tab_v)
        pltpu.sync_copy(g_hbm, g_v)
        pltpu.sync_copy(b_hbm, b_v)
        pltpu.sync_copy(idx_hbm.at[wid], idx_v)

        inv_h = 1.0 / h
        for r in range(v):
            roff = r * h

            def sum_body(j, acc, _roff=roff):
                return acc + tab_v[pl.ds(pl.multiple_of(_roff + j * _LANES, 8), _LANES)]

            acc = lax.fori_loop(0, nsl, sum_body, jnp.zeros((_LANES,), jnp.float32))
            mean = sum(acc[t] for t in range(_LANES)) * inv_h

            def var_body(j, acc2, _roff=roff, _mean=mean):
                d = tab_v[pl.ds(pl.multiple_of(_roff + j * _LANES, 8), _LANES)] - _mean
                return acc2 + d * d

            acc2 = lax.fori_loop(0, nsl, var_body, jnp.zeros((_LANES,), jnp.float32))
            var = sum(acc2[t] for t in range(_LANES)) * inv_h
            inv = _rsqrt16(jnp.full((_LANES,), var + EPS, jnp.float32))

            def nrm_body(j, carry, _roff=roff, _mean=mean, _inv=inv):
                o = pl.multiple_of(_roff + j * _LANES, 8)
                jo = pl.multiple_of(j * _LANES, 8)
                nrm_v[pl.ds(o, _LANES)] = (
                    (tab_v[pl.ds(o, _LANES)] - _mean) * _inv * g_v[pl.ds(jo, _LANES)]
                    + b_v[pl.ds(jo, _LANES)]
                )
                return carry

            lax.fori_loop(0, nsl, nrm_body, 0)

        def group(g, carry):
            goff = pl.multiple_of(g * _LANES, 8)
            idx16 = idx_v[pl.ds(goff, _LANES)]
            gbase = (base + g * _LANES) * h
            for t in range(_LANES):
                src = pl.multiple_of(idx16[t] * h, 8)
                dst = pl.multiple_of(gbase + t * h, 8)
                pltpu.async_copy(
                    nrm_v.at[pl.ds(src, h)], out_hbm.at[pl.ds(dst, h)], sem
                )
            return carry

        lax.fori_loop(0, ngrp, group, 0)

        def drain(i, carry):
            pltpu.make_async_copy(
                nrm_v.at[pl.ds(0, h)],
                out_hbm.at[pl.ds(base * h, h)],
                sem,
            ).wait()
            return carry

        lax.fori_loop(0, bpw, drain, 0)

    return k


def kernel(sentiment_input, table, gamma, beta):
    v, h = table.shape
    idx = sentiment_input.reshape(-1).astype(jnp.int32)
    n_tokens = idx.shape[0]
    fused = _make_fused(n_tokens, v, h)
    out = fused(
        table.reshape(-1),
        gamma.reshape(-1),
        beta.reshape(-1),
        idx.reshape(_NW, n_tokens // _NW),
    )
    return out.reshape(sentiment_input.shape + (h,))


# speedup vs baseline: 2.8733x; 2.8733x over previous
import functools

import jax
import jax.numpy as jnp
from jax import lax
from jax.experimental import pallas as pl
from jax.experimental.pallas import tpu as pltpu
from jax.experimental.pallas import tpu_sc as plsc

HIDDEN = 1024
EPS = 1e-12

_NUM_CORES = 2
_NUM_SUBCORES = 16
_NW = _NUM_CORES * _NUM_SUBCORES
_LANES = 16


def _rsqrt16(s):
    t = jnp.full((_LANES,), 1.0, jnp.float32)
    for _ in range(24):
        t = 0.5 * (t + s / t)
    return 1.0 / t


@functools.lru_cache(maxsize=None)
def _make_fused(n_tokens, v, h):
    bpw = n_tokens // _NW
    ngrp = bpw // _LANES
    nsl = h // _LANES
    mesh = plsc.VectorSubcoreMesh(core_axis_name="c", subcore_axis_name="s")

    @functools.partial(
        pl.kernel,
        mesh=mesh,
        compiler_params=pltpu.CompilerParams(use_tc_tiling_on_sc=True),
        out_type=jax.ShapeDtypeStruct((n_tokens, h), jnp.float32),
        scratch_types=[
            pltpu.VMEM((bpw,), jnp.int32),
            pltpu.VMEM((v * h,), jnp.float32),
            pltpu.VMEM((v * h,), jnp.float32),
            pltpu.VMEM((h,), jnp.float32),
            pltpu.VMEM((h,), jnp.float32),
            pltpu.SemaphoreType.DMA,
        ],
    )
    def k(tab_hbm, g_hbm, b_hbm, idx_hbm, out_hbm, idx_v, tab_v, nrm_v, g_v, b_v, sem):
        wid = lax.axis_index("s") * _NUM_CORES + lax.axis_index("c")
        base = wid * bpw
        pltpu.sync_copy(tab_hbm, tab_v)
        pltpu.sync_copy(g_hbm, g_v)
        pltpu.sync_copy(b_hbm, b_v)
        pltpu.sync_copy(idx_hbm.at[wid], idx_v)

        inv_h = 1.0 / h
        for r in range(v):
            roff = r * h

            def sum_body(j, acc, _roff=roff):
                return acc + tab_v[pl.ds(pl.multiple_of(_roff + j * _LANES, 8), _LANES)]

            acc = lax.fori_loop(0, nsl, sum_body, jnp.zeros((_LANES,), jnp.float32))
            mean = sum(acc[t] for t in range(_LANES)) * inv_h

            def var_body(j, acc2, _roff=roff, _mean=mean):
                d = tab_v[pl.ds(pl.multiple_of(_roff + j * _LANES, 8), _LANES)] - _mean
                return acc2 + d * d

            acc2 = lax.fori_loop(0, nsl, var_body, jnp.zeros((_LANES,), jnp.float32))
            var = sum(acc2[t] for t in range(_LANES)) * inv_h
            inv = _rsqrt16(jnp.full((_LANES,), var + EPS, jnp.float32))

            def nrm_body(j, carry, _roff=roff, _mean=mean, _inv=inv):
                o = pl.multiple_of(_roff + j * _LANES, 8)
                jo = pl.multiple_of(j * _LANES, 8)
                nrm_v[pl.ds(o, _LANES)] = (
                    (tab_v[pl.ds(o, _LANES)] - _mean) * _inv * g_v[pl.ds(jo, _LANES)]
                    + b_v[pl.ds(jo, _LANES)]
                )
                return carry

            lax.fori_loop(0, nsl, nrm_body, 0)

        def group(g, carry):
            goff = pl.multiple_of(g * _LANES, 8)
            idx16 = idx_v[pl.ds(goff, _LANES)]
            gbase = base + g * _LANES
            for t in range(_LANES):
                src = pl.multiple_of(idx16[t] * h, 8)
                pltpu.async_copy(
                    nrm_v.at[pl.ds(src, h)],
                    out_hbm.at[gbase + t],
                    sem,
                )
            return carry

        lax.fori_loop(0, ngrp, group, 0)

        def drain(i, carry):
            pltpu.make_async_copy(
                nrm_v.at[pl.ds(0, h)],
                out_hbm.at[base],
                sem,
            ).wait()
            return carry

        lax.fori_loop(0, bpw, drain, 0)

    return k


def kernel(sentiment_input, table, gamma, beta):
    v, h = table.shape
    idx = sentiment_input.reshape(-1).astype(jnp.int32)
    n_tokens = idx.shape[0]
    fused = _make_fused(n_tokens, v, h)
    out = fused(
        table.reshape(-1),
        gamma.reshape(-1),
        beta.reshape(-1),
        idx.reshape(_NW, n_tokens // _NW),
    )
    return out.reshape(sentiment_input.shape + (h,))


# speedup vs baseline: 2.8855x; 1.0042x over previous
import functools

import jax
import jax.numpy as jnp
from jax import lax
from jax.experimental import pallas as pl
from jax.experimental.pallas import tpu as pltpu
from jax.experimental.pallas import tpu_sc as plsc

HIDDEN = 1024
EPS = 1e-12

_NUM_CORES = 2
_NUM_SUBCORES = 16
_NW = _NUM_CORES * _NUM_SUBCORES
_LANES = 16


def _rsqrt16(s):
    t = jnp.full((_LANES,), 1.0, jnp.float32)
    for _ in range(16):
        t = 0.5 * (t + s / t)
    return 1.0 / t


@functools.lru_cache(maxsize=None)
def _make_fused(batch, seq, v, h):
    n_tokens = batch * seq
    bpw = n_tokens // _NW
    wpb = seq // bpw
    ngrp = bpw // _LANES
    nsl = h // _LANES
    mesh = plsc.VectorSubcoreMesh(core_axis_name="c", subcore_axis_name="s")

    @functools.partial(
        pl.kernel,
        mesh=mesh,
        compiler_params=pltpu.CompilerParams(use_tc_tiling_on_sc=True),
        out_type=jax.ShapeDtypeStruct((n_tokens, h), jnp.float32),
        scratch_types=[
            pltpu.VMEM((bpw,), jnp.int32),
            pltpu.VMEM((v * h,), jnp.float32),
            pltpu.VMEM((v * h,), jnp.float32),
            pltpu.VMEM((h,), jnp.float32),
            pltpu.VMEM((h,), jnp.float32),
            pltpu.SemaphoreType.DMA,
        ],
    )
    def k(tab_hbm, g_hbm, b_hbm, idx_hbm, out_hbm, idx_v, tab_v, nrm_v, g_v, b_v, sem):
        wid = lax.axis_index("s") * _NUM_CORES + lax.axis_index("c")
        base = wid * bpw
        pltpu.sync_copy(tab_hbm, tab_v)
        pltpu.sync_copy(g_hbm, g_v)
        pltpu.sync_copy(b_hbm, b_v)
        pltpu.sync_copy(
            idx_hbm.at[wid // wpb, pl.ds((wid % wpb) * bpw, bpw)], idx_v
        )

        inv_h = 1.0 / h
        for r in range(v):
            roff = r * h

            def sum_body(j, acc, _roff=roff):
                return acc + tab_v[pl.ds(pl.multiple_of(_roff + j * _LANES, 8), _LANES)]

            acc = lax.fori_loop(0, nsl, sum_body, jnp.zeros((_LANES,), jnp.float32))
            mean = sum(acc[t] for t in range(_LANES)) * inv_h

            def var_body(j, acc2, _roff=roff, _mean=mean):
                d = tab_v[pl.ds(pl.multiple_of(_roff + j * _LANES, 8), _LANES)] - _mean
                return acc2 + d * d

            acc2 = lax.fori_loop(0, nsl, var_body, jnp.zeros((_LANES,), jnp.float32))
            var = sum(acc2[t] for t in range(_LANES)) * inv_h
            inv = _rsqrt16(jnp.full((_LANES,), var + EPS, jnp.float32))

            def nrm_body(j, carry, _roff=roff, _mean=mean, _inv=inv):
                o = pl.multiple_of(_roff + j * _LANES, 8)
                jo = pl.multiple_of(j * _LANES, 8)
                nrm_v[pl.ds(o, _LANES)] = (
                    (tab_v[pl.ds(o, _LANES)] - _mean) * _inv * g_v[pl.ds(jo, _LANES)]
                    + b_v[pl.ds(jo, _LANES)]
                )
                return carry

            lax.fori_loop(0, nsl, nrm_body, 0)

        def group(g, carry):
            goff = pl.multiple_of(g * _LANES, 8)
            idx16 = idx_v[pl.ds(goff, _LANES)]
            gbase = base + g * _LANES
            for t in range(_LANES):
                src = pl.multiple_of(idx16[t] * h, 8)
                pltpu.async_copy(
                    nrm_v.at[pl.ds(src, h)],
                    out_hbm.at[gbase + t],
                    sem,
                )
            return carry

        lax.fori_loop(0, ngrp, group, 0)

        def drain(i, carry):
            pltpu.make_async_copy(
                nrm_v.at[pl.ds(0, h)],
                out_hbm.at[base],
                sem,
            ).wait()
            return carry

        lax.fori_loop(0, bpw, drain, 0)

    return k


def kernel(sentiment_input, table, gamma, beta):
    v, h = table.shape
    batch, seq = sentiment_input.shape
    fused = _make_fused(batch, seq, v, h)
    out = fused(
        table.reshape(-1),
        gamma.reshape(-1),
        beta.reshape(-1),
        sentiment_input.astype(jnp.int32),
    )
    return out.reshape(batch, seq, h)


# speedup vs baseline: 2.9415x; 1.0194x over previous
import functools

import jax
import jax.numpy as jnp
from jax import lax
from jax.experimental import pallas as pl
from jax.experimental.pallas import tpu as pltpu
from jax.experimental.pallas import tpu_sc as plsc

HIDDEN = 1024
EPS = 1e-12

_NUM_CORES = 2
_NUM_SUBCORES = 16
_NW = _NUM_CORES * _NUM_SUBCORES
_LANES = 16


def _rsqrt16(s):
    t = jnp.full((_LANES,), 1.0, jnp.float32)
    for _ in range(16):
        t = 0.5 * (t + s / t)
    return 1.0 / t


@functools.lru_cache(maxsize=None)
def _make_fused(batch, seq, v, h):
    n_tokens = batch * seq
    bpw = n_tokens // _NW
    wpb = seq // bpw
    ngrp = bpw // _LANES
    nsl = h // _LANES
    mesh = plsc.VectorSubcoreMesh(core_axis_name="c", subcore_axis_name="s")

    @functools.partial(
        pl.kernel,
        mesh=mesh,
        compiler_params=pltpu.CompilerParams(use_tc_tiling_on_sc=True),
        out_type=jax.ShapeDtypeStruct((n_tokens, h), jnp.float32),
        scratch_types=[
            pltpu.VMEM((bpw,), jnp.int32),
            pltpu.VMEM((v * h,), jnp.float32),
            pltpu.VMEM((v * h,), jnp.float32),
            pltpu.VMEM((h,), jnp.float32),
            pltpu.VMEM((h,), jnp.float32),
            pltpu.SemaphoreType.DMA,
            pltpu.SemaphoreType.DMA,
        ],
    )
    def k(tab_hbm, g_hbm, b_hbm, idx_hbm, out_hbm, idx_v, tab_v, nrm_v, g_v, b_v, sem, isem):
        wid = lax.axis_index("s") * _NUM_CORES + lax.axis_index("c")
        base = wid * bpw
        idx_cp = pltpu.async_copy(
            idx_hbm.at[wid // wpb, pl.ds((wid % wpb) * bpw, bpw)], idx_v, isem
        )
        pltpu.sync_copy(tab_hbm, tab_v)
        pltpu.sync_copy(g_hbm, g_v)
        pltpu.sync_copy(b_hbm, b_v)

        inv_h = 1.0 / h
        for r in range(v):
            roff = r * h

            def sum_body(j, accs, _roff=roff):
                acc, acc2 = accs
                x = tab_v[pl.ds(pl.multiple_of(_roff + j * _LANES, 8), _LANES)]
                return acc + x, acc2 + x * x

            zero = jnp.zeros((_LANES,), jnp.float32)
            acc, acc2 = lax.fori_loop(0, nsl, sum_body, (zero, zero))
            mean = sum(acc[t] for t in range(_LANES)) * inv_h
            msq = sum(acc2[t] for t in range(_LANES)) * inv_h
            var = msq - mean * mean
            inv = _rsqrt16(jnp.full((_LANES,), var + EPS, jnp.float32))

            def nrm_body(j, carry, _roff=roff, _mean=mean, _inv=inv):
                o = pl.multiple_of(_roff + j * _LANES, 8)
                jo = pl.multiple_of(j * _LANES, 8)
                nrm_v[pl.ds(o, _LANES)] = (
                    (tab_v[pl.ds(o, _LANES)] - _mean) * _inv * g_v[pl.ds(jo, _LANES)]
                    + b_v[pl.ds(jo, _LANES)]
                )
                return carry

            lax.fori_loop(0, nsl, nrm_body, 0)

        idx_cp.wait()

        def group(g, carry):
            goff = pl.multiple_of(g * _LANES, 8)
            idx16 = idx_v[pl.ds(goff, _LANES)]
            gbase = base + g * _LANES
            for t in range(_LANES):
                src = pl.multiple_of(idx16[t] * h, 8)
                pltpu.async_copy(
                    nrm_v.at[pl.ds(src, h)],
                    out_hbm.at[gbase + t],
                    sem,
                )
            return carry

        lax.fori_loop(0, ngrp, group, 0)

        def drain(i, carry):
            pltpu.make_async_copy(
                nrm_v.at[pl.ds(0, h)],
                out_hbm.at[base],
                sem,
            ).wait()
            return carry

        lax.fori_loop(0, bpw, drain, 0)

    return k


def kernel(sentiment_input, table, gamma, beta):
    v, h = table.shape
    batch, seq = sentiment_input.shape
    fused = _make_fused(batch, seq, v, h)
    out = fused(
        table.reshape(-1),
        gamma.reshape(-1),
        beta.reshape(-1),
        sentiment_input.astype(jnp.int32),
    )
    return out.reshape(batch, seq, h)
